# baseline (speedup 1.0000x reference)
import jax
import jax.numpy as jnp
from jax.experimental import pallas as pl
from jax.experimental.pallas import tpu as pltpu

_BM = 400


def _gcn_body(x_ref, adj_l_ref, adj_r_ref, w_ref, b_ref, o_ref):
    i = pl.program_id(0)
    d = x_ref.shape[1]
    nk = adj_l_ref.shape[-1]
    agg = jnp.dot(adj_l_ref[:, 0, 0, :].astype(jnp.bfloat16),
                  x_ref[0:nk, :].astype(jnp.bfloat16),
                  preferred_element_type=jnp.float32)
    agg += jnp.dot(adj_r_ref[:, 0, 0, :].astype(jnp.bfloat16),
                   x_ref[nk:2 * nk, :].astype(jnp.bfloat16),
                   preferred_element_type=jnp.float32)
    xs = x_ref[pl.ds(i * _BM, _BM), :]
    h = (jnp.dot(xs, w_ref[0:d, :], preferred_element_type=jnp.float32)
         + jnp.dot(agg, w_ref[d:2 * d, :], preferred_element_type=jnp.float32)
         + b_ref[...])
    o_ref[...] = jnp.maximum(h, 0.0)


def kernel(x, adj, W, b):
    n, d = x.shape
    nh = W.shape[1]
    half = n // 2
    adj4 = adj.reshape(n, 2, 1, half)
    grid = (n // _BM,)
    return pl.pallas_call(
        _gcn_body,
        grid=grid,
        in_specs=[
            pl.BlockSpec((n, d), lambda i: (0, 0)),
            pl.BlockSpec((_BM, 1, 1, half), lambda i: (i, 0, 0, 0)),
            pl.BlockSpec((_BM, 1, 1, half), lambda i: (i, 1, 0, 0)),
            pl.BlockSpec((2 * d, nh), lambda i: (0, 0)),
            pl.BlockSpec((1, nh), lambda i: (0, 0)),
        ],
        out_specs=pl.BlockSpec((_BM, nh), lambda i: (i, 0)),
        out_shape=jax.ShapeDtypeStruct((n, nh), jnp.float32),
        compiler_params=pltpu.CompilerParams(
            dimension_semantics=("arbitrary",),
        ),
    )(x, adj4, adj4, W, b.reshape(1, nh))


# speedup vs baseline: 25.6231x; 25.6231x over previous
import jax
import jax.numpy as jnp
from jax.experimental import pallas as pl
from jax.experimental.pallas import tpu as pltpu

_BM = 200


def _gcn_body(x_ref, adj_a_ref, adj_b_ref, w_ref, b_ref, o_ref):
    i = pl.program_id(0)
    d = x_ref.shape[1]
    half = x_ref.shape[0] // 2
    xb = x_ref[...].astype(jnp.bfloat16)
    w1 = w_ref[0:d, :]
    w2 = w_ref[d:2 * d, :]
    for off, ref in ((0, adj_a_ref), (half, adj_b_ref)):
        agg = jnp.dot(ref[...].astype(jnp.bfloat16), xb,
                      preferred_element_type=jnp.float32)
        xs = x_ref[pl.ds(off + i * _BM, _BM), :]
        h = (jnp.dot(xs, w1, preferred_element_type=jnp.float32)
             + jnp.dot(agg, w2, preferred_element_type=jnp.float32)
             + b_ref[...])
        o_ref[pl.ds(off + i * _BM, _BM), :] = jnp.maximum(h, 0.0)


def kernel(x, adj, W, b):
    n, d = x.shape
    nh = W.shape[1]
    nb = n // 2 // _BM
    grid = (nb,)
    return pl.pallas_call(
        _gcn_body,
        grid=grid,
        in_specs=[
            pl.BlockSpec((n, d), lambda i: (0, 0)),
            pl.BlockSpec((_BM, n), lambda i: (i, 0)),
            pl.BlockSpec((_BM, n), lambda i: (i + nb, 0)),
            pl.BlockSpec((2 * d, nh), lambda i: (0, 0)),
            pl.BlockSpec((1, nh), lambda i: (0, 0)),
        ],
        out_specs=pl.BlockSpec((n, nh), lambda i: (0, 0)),
        out_shape=jax.ShapeDtypeStruct((n, nh), jnp.float32),
        compiler_params=pltpu.CompilerParams(
            dimension_semantics=("arbitrary",),
        ),
    )(x, adj, adj, W, b.reshape(1, nh))


# speedup vs baseline: 26.0829x; 1.0179x over previous
import jax
import jax.numpy as jnp
from jax.experimental import pallas as pl
from jax.experimental.pallas import tpu as pltpu

_BM = 400


def _gcn_body(x_ref, adj_ref, w_ref, b_ref, o_ref):
    i = pl.program_id(0)
    d = x_ref.shape[1]
    agg = jnp.dot(adj_ref[...].astype(jnp.bfloat16),
                  x_ref[...].astype(jnp.bfloat16),
                  preferred_element_type=jnp.float32)
    xs = x_ref[pl.ds(i * _BM, _BM), :]
    h = (jnp.dot(xs, w_ref[0:d, :], preferred_element_type=jnp.float32)
         + jnp.dot(agg, w_ref[d:2 * d, :], preferred_element_type=jnp.float32)
         + b_ref[...])
    o_ref[pl.ds(i * _BM, _BM), :] = jnp.maximum(h, 0.0)


def kernel(x, adj, W, b):
    n, d = x.shape
    nh = W.shape[1]
    grid = (n // _BM,)
    return pl.pallas_call(
        _gcn_body,
        grid=grid,
        in_specs=[
            pl.BlockSpec((n, d), lambda i: (0, 0)),
            pl.BlockSpec((_BM, n), lambda i: (i, 0)),
            pl.BlockSpec((2 * d, nh), lambda i: (0, 0)),
            pl.BlockSpec((1, nh), lambda i: (0, 0)),
        ],
        out_specs=pl.BlockSpec((n, nh), lambda i: (0, 0)),
        out_shape=jax.ShapeDtypeStruct((n, nh), jnp.float32),
        compiler_params=pltpu.CompilerParams(
            dimension_semantics=("arbitrary",),
        ),
    )(x, adj, W, b.reshape(1, nh))


# speedup vs baseline: 26.2113x; 1.0049x over previous
import jax
import jax.numpy as jnp
from jax.experimental import pallas as pl
from jax.experimental.pallas import tpu as pltpu

_BM = 400


def _gcn_body(x_ref, adj_ref, w_ref, b_ref, o_ref):
    i = pl.program_id(0)
    d = x_ref.shape[1]
    agg = jnp.dot(adj_ref[...].astype(jnp.bfloat16),
                  x_ref[...].astype(jnp.bfloat16),
                  preferred_element_type=jnp.float32)
    xs = x_ref[pl.ds(i * _BM, _BM), :]
    h = (jnp.dot(xs, w_ref[0:d, :], preferred_element_type=jnp.float32)
         + jnp.dot(agg, w_ref[d:2 * d, :], preferred_element_type=jnp.float32)
         + b_ref[...])
    o_ref[...] = jnp.maximum(h, 0.0)


def kernel(x, adj, W, b):
    n, d = x.shape
    nh = W.shape[1]
    grid = (n // _BM,)
    return pl.pallas_call(
        _gcn_body,
        grid=grid,
        in_specs=[
            pl.BlockSpec((n, d), lambda i: (0, 0)),
            pl.BlockSpec((_BM, n), lambda i: (i, 0)),
            pl.BlockSpec((2 * d, nh), lambda i: (0, 0)),
            pl.BlockSpec((1, nh), lambda i: (0, 0)),
        ],
        out_specs=pl.BlockSpec((_BM, nh), lambda i: (i, 0)),
        out_shape=jax.ShapeDtypeStruct((n, nh), jnp.float32),
        compiler_params=pltpu.CompilerParams(
            dimension_semantics=("parallel",),
        ),
    )(x, adj, W, b.reshape(1, nh))
